# baseline (speedup 1.0000x reference)
import jax
import jax.numpy as jnp
from jax import lax
from jax.experimental import pallas as pl
from jax.experimental.pallas import tpu as pltpu

KNN = 32
NEG = -1e30


def _l2n(x):
    ss = jnp.sum(x * x, axis=-1, keepdims=True)
    return x / jnp.maximum(jnp.sqrt(ss), 1e-12)


def _keyify(x):
    s = lax.bitcast_convert_type(x, jnp.int32)
    return s ^ (lax.shift_right_arithmetic(s, 31) & jnp.int32(0x7FFFFFFF))


def _attn_body(scale_ref, q_ref, k_ref, v_ref, maskf_ref, mk_ref, mv_ref,
               o_ref, kn_ref, mkn_ref):
    h = pl.program_id(0)
    qi = pl.program_id(1)

    @pl.when(jnp.logical_and(h == 0, qi == 0))
    def _():
        kn_ref[...] = _l2n(k_ref[0])
        mkn_ref[...] = _l2n(mk_ref[0])

    scale = jnp.exp(jnp.full((1, 1), scale_ref[h], jnp.float32))
    qn = _l2n(q_ref[0, 0])

    smem = lax.dot_general(qn, mkn_ref[...], (((1,), (1,)), ((), ())),
                           preferred_element_type=jnp.float32)
    bq, m = smem.shape

    red = smem
    while red.shape[-1] > 128:
        half = red.shape[-1] // 2
        red = jnp.maximum(red[:, :half], red[:, half:])
    t_lo = jnp.min(red, axis=-1, keepdims=True)
    t_hi = jnp.max(red, axis=-1, keepdims=True)

    def _mid(il, ih):
        return ((il >> 1) + (ih >> 1)) + (il & ih & 1)

    keys = _keyify(smem)

    def _count(mid):
        return jnp.sum((keys >= mid).astype(jnp.int32), axis=-1,
                       keepdims=True)

    il0 = _keyify(t_lo)
    cl0 = jnp.full((bq, 1), m, jnp.int32)
    ih0 = _keyify(t_hi) + 1

    def cond(st):
        il, ih, cl = st
        return jnp.any(jnp.logical_and(_mid(il, ih) > il, cl != KNN))

    def body(st):
        il, ih, cl = st
        mid = _mid(il, ih)
        c = _count(mid)
        ge = c >= KNN
        return (jnp.where(ge, mid, il), jnp.where(ge, ih, mid),
                jnp.where(ge, c, cl))

    il, _, _ = lax.while_loop(cond, body, (il0, ih0, cl0))
    lm = jnp.where(keys >= il, smem * scale, NEG)

    sl = lax.dot_general(qn, kn_ref[...], (((1,), (1,)), ((), ())),
                         preferred_element_type=jnp.float32) * scale
    sl = sl + NEG * (1.0 - maskf_ref[0])[None, :]

    mx = jnp.maximum(t_hi * scale, jnp.max(sl, axis=-1, keepdims=True))
    pm = jnp.exp(lm - mx)
    pll = jnp.exp(sl - mx)
    z = (jnp.sum(pm, axis=-1, keepdims=True) +
         jnp.sum(pll, axis=-1, keepdims=True))
    out = (jnp.dot(pm, mv_ref[0], preferred_element_type=jnp.float32) +
           jnp.dot(pll, v_ref[0], preferred_element_type=jnp.float32)) / z
    o_ref[0, 0] = out


@jax.jit
def kernel(q, k, v, mask, mem_keys, mem_vals, scale_param):
    B, H, S, D = q.shape
    M = mem_keys.shape[1]
    BQ = 128 if S % 128 == 0 else S
    maskf = mask.astype(jnp.float32)
    scales = scale_param.reshape(H)

    return pl.pallas_call(
        _attn_body,
        grid=(H, S // BQ),
        in_specs=[
            pl.BlockSpec((H,), lambda h, i: (0,), memory_space=pltpu.SMEM),
            pl.BlockSpec((1, 1, BQ, D), lambda h, i: (0, h, i, 0)),
            pl.BlockSpec((1, S, D), lambda h, i: (0, 0, 0)),
            pl.BlockSpec((1, S, D), lambda h, i: (0, 0, 0)),
            pl.BlockSpec((1, S), lambda h, i: (0, 0)),
            pl.BlockSpec((1, M, D), lambda h, i: (0, 0, 0)),
            pl.BlockSpec((1, M, D), lambda h, i: (0, 0, 0)),
        ],
        out_specs=pl.BlockSpec((1, 1, BQ, D), lambda h, i: (0, h, i, 0)),
        out_shape=jax.ShapeDtypeStruct((B, H, S, D), jnp.float32),
        scratch_shapes=[
            pltpu.VMEM((S, D), jnp.float32),
            pltpu.VMEM((M, D), jnp.float32),
        ],
    )(scales, q, k, v, maskf, mem_keys, mem_vals)


# speedup vs baseline: 1.0731x; 1.0731x over previous
import jax
import jax.numpy as jnp
from jax import lax
from jax.experimental import pallas as pl
from jax.experimental.pallas import tpu as pltpu

KNN = 32
NEG = -1e30


def _l2n(x):
    ss = jnp.sum(x * x, axis=-1, keepdims=True)
    return x / jnp.maximum(jnp.sqrt(ss), 1e-12)


def _keyify(x):
    s = lax.bitcast_convert_type(x, jnp.int32)
    return s ^ (lax.shift_right_arithmetic(s, 31) & jnp.int32(0x7FFFFFFF))


def _attn_body(scale_ref, q_ref, k_ref, v_ref, maskf_ref, mk_ref, mv_ref,
               o_ref, kn_ref, mkn_ref):
    h = pl.program_id(0)
    qi = pl.program_id(1)

    @pl.when(jnp.logical_and(h == 0, qi == 0))
    def _():
        kn_ref[...] = _l2n(k_ref[0])
        mkn_ref[...] = _l2n(mk_ref[0])

    scale = jnp.exp(jnp.full((1, 1), scale_ref[h], jnp.float32))
    qn = _l2n(q_ref[0, 0])

    smem = lax.dot_general(qn, mkn_ref[...], (((1,), (1,)), ((), ())),
                           preferred_element_type=jnp.float32)
    bq, m = smem.shape

    red = smem
    while red.shape[-1] > 128:
        half = red.shape[-1] // 2
        red = jnp.maximum(red[:, :half], red[:, half:])
    t_lo = jnp.min(red, axis=-1, keepdims=True)
    t_hi = jnp.max(red, axis=-1, keepdims=True)

    def _mid(il, ih):
        return ((il >> 1) + (ih >> 1)) + (il & ih & 1)

    keys = _keyify(smem)

    def _count(mid):
        return jnp.sum((keys >= mid).astype(jnp.int32), axis=-1,
                       keepdims=True)

    sig = jnp.sqrt(jnp.sum(smem * smem, axis=-1, keepdims=True) *
                   (1.0 / m))
    t_est = sig * 2.40
    ka = _keyify(t_est * 0.85)
    kb = _keyify(t_est * 1.15)
    ca = _count(ka)
    cb = _count(kb)
    ok_a = ca >= KNN
    ok_b = cb < KNN
    il0 = jnp.where(ok_a, ka, _keyify(t_lo))
    cl0 = jnp.where(ok_a, ca, jnp.full_like(ca, m))
    ih0 = jnp.where(ok_b, kb, _keyify(t_hi) + 1)

    def cond(st):
        il, ih, cl = st
        return jnp.any(jnp.logical_and(_mid(il, ih) > il, cl != KNN))

    def body(st):
        il, ih, cl = st
        mid = _mid(il, ih)
        c = _count(mid)
        ge = c >= KNN
        return (jnp.where(ge, mid, il), jnp.where(ge, ih, mid),
                jnp.where(ge, c, cl))

    il, _, _ = lax.while_loop(cond, body, (il0, ih0, cl0))
    lm = jnp.where(keys >= il, smem * scale, NEG)

    sl = lax.dot_general(qn, kn_ref[...], (((1,), (1,)), ((), ())),
                         preferred_element_type=jnp.float32) * scale
    sl = sl + NEG * (1.0 - maskf_ref[0])[None, :]

    mx = jnp.maximum(t_hi * scale, jnp.max(sl, axis=-1, keepdims=True))
    pm = jnp.exp(lm - mx)
    pll = jnp.exp(sl - mx)
    z = (jnp.sum(pm, axis=-1, keepdims=True) +
         jnp.sum(pll, axis=-1, keepdims=True))
    out = (jnp.dot(pm, mv_ref[0], preferred_element_type=jnp.float32) +
           jnp.dot(pll, v_ref[0], preferred_element_type=jnp.float32)) / z
    o_ref[0, 0] = out


@jax.jit
def kernel(q, k, v, mask, mem_keys, mem_vals, scale_param):
    B, H, S, D = q.shape
    M = mem_keys.shape[1]
    BQ = 256 if S % 256 == 0 else S
    maskf = mask.astype(jnp.float32)
    scales = scale_param.reshape(H)

    return pl.pallas_call(
        _attn_body,
        grid=(H, S // BQ),
        in_specs=[
            pl.BlockSpec((H,), lambda h, i: (0,), memory_space=pltpu.SMEM),
            pl.BlockSpec((1, 1, BQ, D), lambda h, i: (0, h, i, 0)),
            pl.BlockSpec((1, S, D), lambda h, i: (0, 0, 0)),
            pl.BlockSpec((1, S, D), lambda h, i: (0, 0, 0)),
            pl.BlockSpec((1, S), lambda h, i: (0, 0)),
            pl.BlockSpec((1, M, D), lambda h, i: (0, 0, 0)),
            pl.BlockSpec((1, M, D), lambda h, i: (0, 0, 0)),
        ],
        out_specs=pl.BlockSpec((1, 1, BQ, D), lambda h, i: (0, h, i, 0)),
        out_shape=jax.ShapeDtypeStruct((B, H, S, D), jnp.float32),
        scratch_shapes=[
            pltpu.VMEM((S, D), jnp.float32),
            pltpu.VMEM((M, D), jnp.float32),
        ],
    )(scales, q, k, v, maskf, mem_keys, mem_vals)


# speedup vs baseline: 1.1033x; 1.0281x over previous
import jax
import jax.numpy as jnp
from jax import lax
from jax.experimental import pallas as pl
from jax.experimental.pallas import tpu as pltpu

KNN = 32
NEG = -1e30


def _l2n(x):
    ss = jnp.sum(x * x, axis=-1, keepdims=True)
    return x / jnp.maximum(jnp.sqrt(ss), 1e-12)


def _keyify(x):
    s = lax.bitcast_convert_type(x, jnp.int32)
    return s ^ (lax.shift_right_arithmetic(s, 31) & jnp.int32(0x7FFFFFFF))


def _attn_body(scale_ref, q_ref, k_ref, v_ref, maskf_ref, mk_ref, mv_ref,
               o_ref, kn_ref, mkn_ref):
    h = pl.program_id(0)
    qi = pl.program_id(1)

    @pl.when(jnp.logical_and(h == 0, qi == 0))
    def _():
        kn_ref[...] = _l2n(k_ref[0])
        mkn_ref[...] = _l2n(mk_ref[0])

    scale = jnp.exp(jnp.full((1, 1), scale_ref[h], jnp.float32))
    qn = _l2n(q_ref[0, 0])

    smem = lax.dot_general(qn, mkn_ref[...], (((1,), (1,)), ((), ())),
                           preferred_element_type=jnp.float32)
    bq, m = smem.shape

    red = smem
    while red.shape[-1] > 128:
        half = red.shape[-1] // 2
        red = jnp.maximum(red[:, :half], red[:, half:])
    t_lo = jnp.min(red, axis=-1, keepdims=True)
    t_hi = jnp.max(red, axis=-1, keepdims=True)

    def _mid(il, ih):
        return ((il >> 1) + (ih >> 1)) + (il & ih & 1)

    keys = _keyify(smem)

    def _count(mid):
        return jnp.sum((keys >= mid).astype(jnp.int32), axis=-1,
                       keepdims=True)

    sig = jnp.sqrt(jnp.sum(smem * smem, axis=-1, keepdims=True) *
                   (1.0 / m))
    t_est = sig * 2.40
    ka = _keyify(t_est * 0.85)
    kb = _keyify(t_est * 1.15)
    ca = _count(ka)
    cb = _count(kb)
    ok_a = ca >= KNN
    ok_b = cb < KNN
    il0 = jnp.where(ok_a, ka, _keyify(t_lo))
    cl0 = jnp.where(ok_a, ca, jnp.full_like(ca, m))
    ih0 = jnp.where(ok_b, kb, _keyify(t_hi) + 1)

    def cond(st):
        il, ih, cl = st
        return jnp.any(jnp.logical_and(_mid(il, ih) > il, cl != KNN))

    def body(st):
        il, ih, cl = st
        mid = _mid(il, ih)
        c = _count(mid)
        ge = c >= KNN
        return (jnp.where(ge, mid, il), jnp.where(ge, ih, mid),
                jnp.where(ge, c, cl))

    il, _, _ = lax.while_loop(cond, body, (il0, ih0, cl0))
    lm = jnp.where(keys >= il, smem * scale, NEG)

    sl = lax.dot_general(qn, kn_ref[...], (((1,), (1,)), ((), ())),
                         preferred_element_type=jnp.float32) * scale
    sl = sl + NEG * (1.0 - maskf_ref[0])[None, :]

    mx = jnp.maximum(t_hi * scale, jnp.max(sl, axis=-1, keepdims=True))
    pm = jnp.exp(lm - mx)
    pll = jnp.exp(sl - mx)
    z = (jnp.sum(pm, axis=-1, keepdims=True) +
         jnp.sum(pll, axis=-1, keepdims=True))
    out = (jnp.dot(pm, mv_ref[0], preferred_element_type=jnp.float32) +
           jnp.dot(pll, v_ref[0], preferred_element_type=jnp.float32)) / z
    o_ref[0, 0] = out


@jax.jit
def kernel(q, k, v, mask, mem_keys, mem_vals, scale_param):
    B, H, S, D = q.shape
    M = mem_keys.shape[1]
    BQ = 512 if S % 512 == 0 else S
    maskf = mask.astype(jnp.float32)
    scales = scale_param.reshape(H)

    return pl.pallas_call(
        _attn_body,
        grid=(H, S // BQ),
        in_specs=[
            pl.BlockSpec((H,), lambda h, i: (0,), memory_space=pltpu.SMEM),
            pl.BlockSpec((1, 1, BQ, D), lambda h, i: (0, h, i, 0)),
            pl.BlockSpec((1, S, D), lambda h, i: (0, 0, 0)),
            pl.BlockSpec((1, S, D), lambda h, i: (0, 0, 0)),
            pl.BlockSpec((1, S), lambda h, i: (0, 0)),
            pl.BlockSpec((1, M, D), lambda h, i: (0, 0, 0)),
            pl.BlockSpec((1, M, D), lambda h, i: (0, 0, 0)),
        ],
        out_specs=pl.BlockSpec((1, 1, BQ, D), lambda h, i: (0, h, i, 0)),
        out_shape=jax.ShapeDtypeStruct((B, H, S, D), jnp.float32),
        scratch_shapes=[
            pltpu.VMEM((S, D), jnp.float32),
            pltpu.VMEM((M, D), jnp.float32),
        ],
    )(scales, q, k, v, maskf, mem_keys, mem_vals)
